# Initial kernel scaffold; baseline (speedup 1.0000x reference)
import jax
import jax.numpy as jnp
from jax.experimental import pallas as pl


def kernel(cls_out_0, cls_out_1, cls_out_2, cls_out_3, cls_out_4, box_out_0, box_out_1, box_out_2, box_out_3, box_out_4):
    raise NotImplementedError("write your pallas kernel here")


# speedup vs baseline: 11.1709x; 11.1709x over previous
import jax
import jax.numpy as jnp
from jax import lax
from jax.experimental import pallas as pl
from jax.experimental.pallas import tpu as pltpu
from jax.experimental.pallas import tpu_sc as plsc

_C = 90
_SIZES = (64, 32, 16, 8, 4)
_B = 8
_K = 5000
_N = tuple(810 * s * s for s in _SIZES)
_A_OFF = (0, 36864, 46080, 48384, 48960)
_TOT_ANCH = 49104
_NW = 32
_SLOT = 1024
_NBIN = 4096
_KPAD = 5120
_GCHUNK = (_B * _KPAD) // _NW

_F = tuple(-(-(n // _NW) // 16) * 16 for n in _N)
_CHUNKS = ((34560, 3), (25920, 1), (6480, 1), (1632, 1), (416, 1))
_LG = tuple((s * s).bit_length() - 1 for s in _SIZES)
_BUF = 34560

_MESH = plsc.VectorSubcoreMesh(core_axis_name="c", subcore_axis_name="s")
_CP = pltpu.CompilerParams(needs_layout_passes=False)


def _sortable_u32(x_f32):
    b = lax.bitcast_convert_type(x_f32, jnp.int32)
    m = (b >> 31) | jnp.full((16,), jnp.int32(-(2 ** 31)), jnp.int32)
    return lax.bitcast_convert_type(b ^ m, jnp.uint32)


def _wid():
    return lax.axis_index("c") * 16 + lax.axis_index("s")


def _hist_body(c0, c1, c2, c3, c4, hist_out, buf, hist):
    refs = (c0, c1, c2, c3, c4)
    wid = _wid()
    iota = lax.iota(jnp.int32, 16)
    ones = jnp.ones((16,), jnp.int32)
    zeros = jnp.zeros((16,), jnp.int32)

    @pl.loop(0, _B)
    def _batch(b):
        @pl.loop(0, _NBIN // 16)
        def _zero(j):
            hist[pl.ds(j * 16, 16)] = zeros

        for l in range(5):
            n = _N[l]
            chunk, nch = _CHUNKS[l]
            cov_lo_w = wid * _F[l]
            cov_hi_w = jnp.minimum(cov_lo_w + _F[l], n)

            @pl.loop(0, nch)
            def _chunk(k, n=n, chunk=chunk, cov_lo_w=cov_lo_w,
                       cov_hi_w=cov_hi_w, cref=refs[l]):
                lo = cov_lo_w + k * chunk
                hi = jnp.minimum(lo + chunk, cov_hi_w)
                start = jnp.minimum(lo, n - chunk)
                pltpu.sync_copy(cref.at[pl.ds(b * n + start, chunk)],
                                buf.at[pl.ds(0, chunk)])

                @pl.loop(0, chunk // 16)
                def _vec(v):
                    x = buf[pl.ds(v * 16, 16)]
                    gpos = start + v * 16 + iota
                    mask = (gpos >= lo) & (gpos < hi)
                    s = _sortable_u32(x)
                    bin_ = lax.bitcast_convert_type(
                        s >> jnp.full((16,), 20, jnp.uint32), jnp.int32)
                    plsc.addupdate_scatter(hist, [bin_], ones, mask=mask)

        pltpu.sync_copy(hist, hist_out.at[b, wid])


def _compact_body(c0, c1, c2, c3, c4, t32, vals_out, idxs_out,
                  buf, vbuf, ibuf, tvm):
    refs = (c0, c1, c2, c3, c4)
    wid = _wid()
    iota = lax.iota(jnp.int32, 16)
    ones = jnp.ones((16,), jnp.int32)
    zeros = jnp.zeros((16,), jnp.int32)
    ninf = jnp.full((16,), -jnp.inf, jnp.float32)
    imax = jnp.full((16,), jnp.int32(2 ** 31 - 1), jnp.int32)

    pltpu.sync_copy(t32, tvm)

    @pl.loop(0, _B)
    def _batch(b):
        thr = lax.bitcast_convert_type(
            plsc.load_gather(tvm, [jnp.full((16,), 0, jnp.int32) + b]),
            jnp.uint32)

        @pl.loop(0, _SLOT // 16)
        def _fill(j):
            vbuf[pl.ds(j * 16, 16)] = ninf
            ibuf[pl.ds(j * 16, 16)] = imax

        cnt = jnp.zeros((16,), jnp.int32)
        for l in range(5):
            n = _N[l]
            chunk, nch = _CHUNKS[l]
            lg = _LG[l]
            lvl90 = _A_OFF[l] * 90
            s2m1 = (1 << lg) - 1
            cov_lo_w = wid * _F[l]
            cov_hi_w = jnp.minimum(cov_lo_w + _F[l], n)

            @pl.loop(0, nch, init_carry=cnt)
            def _chunk(k, cnt, n=n, chunk=chunk, lg=lg, lvl90=lvl90,
                       s2m1=s2m1, cov_lo_w=cov_lo_w, cov_hi_w=cov_hi_w,
                       cref=refs[l]):
                lo = cov_lo_w + k * chunk
                hi = jnp.minimum(lo + chunk, cov_hi_w)
                start = jnp.minimum(lo, n - chunk)
                pltpu.sync_copy(cref.at[pl.ds(b * n + start, chunk)],
                                buf.at[pl.ds(0, chunk)])

                @pl.loop(0, chunk // 16, init_carry=cnt)
                def _vec(v, cnt):
                    x = buf[pl.ds(v * 16, 16)]
                    gpos = start + v * 16 + iota
                    s = _sortable_u32(x)
                    keep = (gpos >= lo) & (gpos < hi) & (s >= thr)
                    mi = jnp.where(keep, ones, zeros)
                    pos = cnt + plsc.cumsum(mi) - 1
                    keep = keep & (pos < _SLOT)
                    ch = gpos >> lg
                    hw = gpos & s2m1
                    fidx = hw * 810 + ch + lvl90
                    plsc.store_scatter(vbuf, [pos], x, mask=keep)
                    plsc.store_scatter(ibuf, [pos], fidx, mask=keep)
                    return cnt + plsc.all_reduce_population_count(keep)

                return _vec

            cnt = _chunk

        pltpu.sync_copy(vbuf, vals_out.at[b, wid])
        pltpu.sync_copy(ibuf, idxs_out.at[b, wid])


def _gather_body(table, rowids, out, ridx, gidx, rows, obuf, sem):
    base = _wid() * _GCHUNK
    iota = lax.iota(jnp.int32, 16)

    @pl.loop(0, _GCHUNK // 128)
    def _chunk(g):
        pltpu.sync_copy(rowids.at[pl.ds(base + g * 128, 128)], ridx)

        @pl.loop(0, 8)
        def _mkidx(v):
            a = ridx[pl.ds(v * 16, 16)]
            gidx[pl.ds(v * 16, 16)] = a >> 5

        pltpu.async_copy(table.at[gidx], rows, sem).wait()

        @pl.loop(0, 8)
        def _extract(v):
            a = ridx[pl.ds(v * 16, 16)]
            r_local = v * 16 + iota
            col = (a & 31) * 4
            for j in range(4):
                val = plsc.load_gather(rows, [r_local, col + j])
                plsc.store_scatter(obuf, [r_local, iota * 0 + j], val)

        pltpu.sync_copy(obuf, out.at[pl.ds(base + g * 128, 128)])


_pass1 = pl.kernel(
    _hist_body,
    out_type=jax.ShapeDtypeStruct((_B, _NW, _NBIN), jnp.int32),
    mesh=_MESH,
    compiler_params=_CP,
    scratch_types=[pltpu.VMEM((_BUF,), jnp.float32),
                   pltpu.VMEM((_NBIN,), jnp.int32)],
)

_pass2 = pl.kernel(
    _compact_body,
    out_type=(jax.ShapeDtypeStruct((_B, _NW, _SLOT), jnp.float32),
              jax.ShapeDtypeStruct((_B, _NW, _SLOT), jnp.int32)),
    mesh=_MESH,
    compiler_params=_CP,
    scratch_types=[pltpu.VMEM((_BUF,), jnp.float32),
                   pltpu.VMEM((_SLOT,), jnp.float32),
                   pltpu.VMEM((_SLOT,), jnp.int32),
                   pltpu.VMEM((_B,), jnp.int32)],
)

_pass3 = pl.kernel(
    _gather_body,
    out_type=jax.ShapeDtypeStruct((_B * _KPAD, 4), jnp.float32),
    mesh=_MESH,
    compiler_params=_CP,
    scratch_types=[pltpu.VMEM((128,), jnp.int32),
                   pltpu.VMEM((128,), jnp.int32),
                   pltpu.VMEM((128, 128), jnp.float32),
                   pltpu.VMEM((128, 4), jnp.float32),
                   pltpu.SemaphoreType.DMA],
)


def kernel(cls_out_0, cls_out_1, cls_out_2, cls_out_3, cls_out_4,
           box_out_0, box_out_1, box_out_2, box_out_3, box_out_4):
    cls_in = (cls_out_0, cls_out_1, cls_out_2, cls_out_3, cls_out_4)
    box_in = (box_out_0, box_out_1, box_out_2, box_out_3, box_out_4)
    flats = tuple(c.reshape(-1) for c in cls_in)

    hist = _pass1(*flats)
    counts = hist.sum(axis=1, dtype=jnp.int32)
    csum_ge = jnp.cumsum(counts[:, ::-1], axis=1)[:, ::-1]
    binids = lax.broadcasted_iota(jnp.int32, (_B, _NBIN), 1)
    bstar = jnp.max(jnp.where(csum_ge >= _K, binids, 0), axis=1)
    t32 = lax.bitcast_convert_type(bstar.astype(jnp.uint32) << 20, jnp.int32)

    vals, idxs = _pass2(*flats, t32)
    vflat = vals.reshape(_B, _NW * _SLOT)
    iflat = idxs.reshape(_B, _NW * _SLOT)
    bi = lax.bitcast_convert_type(vflat, jnp.int32)
    kasc = bi ^ ((bi >> 31) & jnp.int32(0x7FFFFFFF))
    _, topidx, topval = lax.sort((-kasc, iflat, vflat), dimension=1, num_keys=2)
    topidx = topidx[:, :_K]
    topval = topval[:, :_K]
    anchors = topidx // _C
    classes = topidx % _C

    table = jnp.concatenate(
        [jnp.transpose(bx, (0, 2, 3, 1)).reshape(_B, -1, 4) for bx in box_in],
        axis=1).reshape(-1, 128)
    apad = jnp.concatenate(
        [anchors, jnp.zeros((_B, _KPAD - _K), jnp.int32)], axis=1)
    rowids = (apad + jnp.arange(_B, dtype=jnp.int32)[:, None] * _TOT_ANCH
              ).reshape(-1)
    boxes = _pass3(table, rowids).reshape(_B, _KPAD, 4)[:, :_K, :]

    return (topval[..., None], boxes, anchors, classes)


# speedup vs baseline: 12.3250x; 1.1033x over previous
import jax
import jax.numpy as jnp
from jax import lax
from jax.experimental import pallas as pl
from jax.experimental.pallas import tpu as pltpu
from jax.experimental.pallas import tpu_sc as plsc

_C = 90
_SIZES = (64, 32, 16, 8, 4)
_B = 8
_K = 5000
_N = tuple(810 * s * s for s in _SIZES)
_A_OFF = (0, 36864, 46080, 48384, 48960)
_TOT_ANCH = 49104
_NW = 32
_SLOT = 512
_NBIN = 4096
_KPAD = 5120
_GCHUNK = (_B * _KPAD) // _NW

_F = tuple(-(-(n // _NW) // 16) * 16 for n in _N)
_CHUNKS = ((34560, 3), (25920, 1), (6480, 1), (1632, 1), (416, 1))
_LG = tuple((s * s).bit_length() - 1 for s in _SIZES)
_BUF = 34560

_MESH = plsc.VectorSubcoreMesh(core_axis_name="c", subcore_axis_name="s")
_CP = pltpu.CompilerParams(needs_layout_passes=False)


def _sortable_u32(x_f32):
    b = lax.bitcast_convert_type(x_f32, jnp.int32)
    m = (b >> 31) | jnp.full((16,), jnp.int32(-(2 ** 31)), jnp.int32)
    return lax.bitcast_convert_type(b ^ m, jnp.uint32)


def _wid():
    return lax.axis_index("c") * 16 + lax.axis_index("s")


_SCHED = ((0, 0, 34560), (0, 1, 34560), (0, 2, 34560), (1, 0, 25920),
          (2, 0, 6480), (3, 0, 1632), (4, 0, 416))


def _hist_body(c0, c1, c2, c3, c4, hist_out, bufa, bufb, hist, sema, semb):
    refs = (c0, c1, c2, c3, c4)
    wid = _wid()
    iota = lax.iota(jnp.int32, 16)
    ones = jnp.ones((16,), jnp.int32)
    zeros = jnp.zeros((16,), jnp.int32)
    bufs = (bufa, bufb)
    sems = (sema, semb)
    u20 = jnp.full((16,), 20, jnp.uint32)

    def _issue(b, i):
        l, k, chunk = _SCHED[i]
        n = _N[l]
        lo = wid * _F[l] + k * chunk
        start = jnp.minimum(lo, n - chunk)
        return pltpu.async_copy(refs[l].at[pl.ds(b * n + start, chunk)],
                                bufs[i % 2].at[pl.ds(0, chunk)], sems[i % 2])

    @pl.loop(0, _B)
    def _batch(b):
        @pl.loop(0, _NBIN // 16)
        def _zero(j):
            hist[pl.ds(j * 16, 16)] = zeros

        copies = [_issue(b, 0)]
        for i, (l, k, chunk) in enumerate(_SCHED):
            if i + 1 < len(_SCHED):
                copies.append(_issue(b, i + 1))
            copies[i].wait()
            buf = bufs[i % 2]
            n = _N[l]
            lo = wid * _F[l] + k * chunk
            hi = jnp.minimum(lo + chunk, jnp.minimum((wid + 1) * _F[l], n))
            start = jnp.minimum(lo, n - chunk)
            if l < 3:
                @pl.loop(0, chunk // 16)
                def _vec(v, buf=buf):
                    x = buf[pl.ds(v * 16, 16)]
                    s = _sortable_u32(x)
                    bin_ = lax.bitcast_convert_type(s >> u20, jnp.int32)
                    plsc.addupdate_scatter(hist, [bin_], ones)
            else:
                @pl.loop(0, chunk // 16)
                def _vec(v, buf=buf, lo=lo, hi=hi, start=start):
                    x = buf[pl.ds(v * 16, 16)]
                    gpos = start + v * 16 + iota
                    mask = (gpos >= lo) & (gpos < hi)
                    s = _sortable_u32(x)
                    bin_ = lax.bitcast_convert_type(s >> u20, jnp.int32)
                    plsc.addupdate_scatter(hist, [bin_], ones, mask=mask)

        pltpu.sync_copy(hist, hist_out.at[b, wid])


def _compact_body(c0, c1, c2, c3, c4, t32, vals_out, idxs_out,
                  bufa, bufb, vbuf, ibuf, tvm, sema, semb):
    refs = (c0, c1, c2, c3, c4)
    wid = _wid()
    iota = lax.iota(jnp.int32, 16)
    ones = jnp.ones((16,), jnp.int32)
    zeros = jnp.zeros((16,), jnp.int32)
    ninf = jnp.full((16,), -jnp.inf, jnp.float32)
    imax = jnp.full((16,), jnp.int32(2 ** 31 - 1), jnp.int32)
    bufs = (bufa, bufb)
    sems = (sema, semb)

    pltpu.sync_copy(t32, tvm)

    def _issue(b, i):
        l, k, chunk = _SCHED[i]
        n = _N[l]
        lo = wid * _F[l] + k * chunk
        start = jnp.minimum(lo, n - chunk)
        return pltpu.async_copy(refs[l].at[pl.ds(b * n + start, chunk)],
                                bufs[i % 2].at[pl.ds(0, chunk)], sems[i % 2])

    @pl.loop(0, _B)
    def _batch(b):
        thr = lax.bitcast_convert_type(
            plsc.load_gather(tvm, [jnp.full((16,), 0, jnp.int32) + b]),
            jnp.uint32)

        copies = [_issue(b, 0)]

        @pl.loop(0, _SLOT // 16)
        def _fill(j):
            vbuf[pl.ds(j * 16, 16)] = ninf
            ibuf[pl.ds(j * 16, 16)] = imax

        cnt = jnp.zeros((16,), jnp.int32)
        for i, (l, k, chunk) in enumerate(_SCHED):
            if i + 1 < len(_SCHED):
                copies.append(_issue(b, i + 1))
            copies[i].wait()
            buf = bufs[i % 2]
            n = _N[l]
            lg = _LG[l]
            lvl90 = _A_OFF[l] * 90
            s2m1 = (1 << lg) - 1
            lo = wid * _F[l] + k * chunk
            hi = jnp.minimum(lo + chunk, jnp.minimum((wid + 1) * _F[l], n))
            start = jnp.minimum(lo, n - chunk)
            masked = l >= 3

            @pl.loop(0, chunk // 16, init_carry=cnt)
            def _vec(v, cnt, buf=buf, lo=lo, hi=hi, start=start, lg=lg,
                     lvl90=lvl90, s2m1=s2m1, masked=masked):
                x = buf[pl.ds(v * 16, 16)]
                gpos = start + v * 16 + iota
                s = _sortable_u32(x)
                keep = s >= thr
                if masked:
                    keep = keep & (gpos >= lo) & (gpos < hi)
                mi = jnp.where(keep, ones, zeros)
                pos = cnt + plsc.cumsum(mi) - 1
                keep = keep & (pos < _SLOT)
                ch = gpos >> lg
                hw = gpos & s2m1
                fidx = hw * 810 + ch + lvl90
                plsc.store_scatter(vbuf, [pos], x, mask=keep)
                plsc.store_scatter(ibuf, [pos], fidx, mask=keep)
                return cnt + plsc.all_reduce_population_count(keep)

            cnt = _vec

        pltpu.sync_copy(vbuf, vals_out.at[b, wid])
        pltpu.sync_copy(ibuf, idxs_out.at[b, wid])


def _gather_body(table, rowids, out, ridx, gidx, rows, obuf, sem):
    base = _wid() * _GCHUNK
    iota = lax.iota(jnp.int32, 16)

    @pl.loop(0, _GCHUNK // 128)
    def _chunk(g):
        pltpu.sync_copy(rowids.at[pl.ds(base + g * 128, 128)], ridx)

        @pl.loop(0, 8)
        def _mkidx(v):
            a = ridx[pl.ds(v * 16, 16)]
            gidx[pl.ds(v * 16, 16)] = a >> 5

        pltpu.async_copy(table.at[gidx], rows, sem).wait()

        @pl.loop(0, 8)
        def _extract(v):
            a = ridx[pl.ds(v * 16, 16)]
            r_local = v * 16 + iota
            col = (a & 31) * 4
            for j in range(4):
                val = plsc.load_gather(rows, [r_local, col + j])
                plsc.store_scatter(obuf, [r_local, iota * 0 + j], val)

        pltpu.sync_copy(obuf, out.at[pl.ds(base + g * 128, 128)])


_pass1 = pl.kernel(
    _hist_body,
    out_type=jax.ShapeDtypeStruct((_B, _NW, _NBIN), jnp.int32),
    mesh=_MESH,
    compiler_params=_CP,
    scratch_types=[pltpu.VMEM((_BUF,), jnp.float32),
                   pltpu.VMEM((_BUF,), jnp.float32),
                   pltpu.VMEM((_NBIN,), jnp.int32),
                   pltpu.SemaphoreType.DMA,
                   pltpu.SemaphoreType.DMA],
)

_pass2 = pl.kernel(
    _compact_body,
    out_type=(jax.ShapeDtypeStruct((_B, _NW, _SLOT), jnp.float32),
              jax.ShapeDtypeStruct((_B, _NW, _SLOT), jnp.int32)),
    mesh=_MESH,
    compiler_params=_CP,
    scratch_types=[pltpu.VMEM((_BUF,), jnp.float32),
                   pltpu.VMEM((_BUF,), jnp.float32),
                   pltpu.VMEM((_SLOT,), jnp.float32),
                   pltpu.VMEM((_SLOT,), jnp.int32),
                   pltpu.VMEM((_B,), jnp.int32),
                   pltpu.SemaphoreType.DMA,
                   pltpu.SemaphoreType.DMA],
)

_pass3 = pl.kernel(
    _gather_body,
    out_type=jax.ShapeDtypeStruct((_B * _KPAD, 4), jnp.float32),
    mesh=_MESH,
    compiler_params=_CP,
    scratch_types=[pltpu.VMEM((128,), jnp.int32),
                   pltpu.VMEM((128,), jnp.int32),
                   pltpu.VMEM((128, 128), jnp.float32),
                   pltpu.VMEM((128, 4), jnp.float32),
                   pltpu.SemaphoreType.DMA],
)


def kernel(cls_out_0, cls_out_1, cls_out_2, cls_out_3, cls_out_4,
           box_out_0, box_out_1, box_out_2, box_out_3, box_out_4):
    cls_in = (cls_out_0, cls_out_1, cls_out_2, cls_out_3, cls_out_4)
    box_in = (box_out_0, box_out_1, box_out_2, box_out_3, box_out_4)
    flats = tuple(c.reshape(-1) for c in cls_in)

    hist = _pass1(*flats)
    counts = hist.sum(axis=1, dtype=jnp.int32)
    csum_ge = jnp.cumsum(counts[:, ::-1], axis=1)[:, ::-1]
    binids = lax.broadcasted_iota(jnp.int32, (_B, _NBIN), 1)
    bstar = jnp.max(jnp.where(csum_ge >= _K, binids, 0), axis=1)
    t32 = lax.bitcast_convert_type(bstar.astype(jnp.uint32) << 20, jnp.int32)

    vals, idxs = _pass2(*flats, t32)
    vflat = vals.reshape(_B, _NW * _SLOT)
    iflat = idxs.reshape(_B, _NW * _SLOT)
    bi = lax.bitcast_convert_type(vflat, jnp.int32)
    kasc = bi ^ ((bi >> 31) & jnp.int32(0x7FFFFFFF))
    _, topidx, topval = lax.sort((-kasc, iflat, vflat), dimension=1, num_keys=2)
    topidx = topidx[:, :_K]
    topval = topval[:, :_K]
    anchors = topidx // _C
    classes = topidx % _C

    table = jnp.concatenate(
        [jnp.transpose(bx, (0, 2, 3, 1)).reshape(_B, -1, 4) for bx in box_in],
        axis=1).reshape(-1, 128)
    apad = jnp.concatenate(
        [anchors, jnp.zeros((_B, _KPAD - _K), jnp.int32)], axis=1)
    rowids = (apad + jnp.arange(_B, dtype=jnp.int32)[:, None] * _TOT_ANCH
              ).reshape(-1)
    boxes = _pass3(table, rowids).reshape(_B, _KPAD, 4)[:, :_K, :]

    return (topval[..., None], boxes, anchors, classes)


# speedup vs baseline: 12.4471x; 1.0099x over previous
import jax
import jax.numpy as jnp
from jax import lax
from jax.experimental import pallas as pl
from jax.experimental.pallas import tpu as pltpu
from jax.experimental.pallas import tpu_sc as plsc

_C = 90
_SIZES = (64, 32, 16, 8, 4)
_B = 8
_K = 5000
_N = tuple(810 * s * s for s in _SIZES)
_A_OFF = (0, 36864, 46080, 48384, 48960)
_TOT_ANCH = 49104
_NW = 32
_SLOT = 512
_NBIN = 4096
_KPAD = 5120
_GCHUNK = (_B * _KPAD) // _NW

_F = tuple(-(-(n // _NW) // 16) * 16 for n in _N)
_CHUNKS = ((34560, 3), (25920, 1), (6480, 1), (1632, 1), (416, 1))
_LG = tuple((s * s).bit_length() - 1 for s in _SIZES)
_BUF = 34560

_MESH = plsc.VectorSubcoreMesh(core_axis_name="c", subcore_axis_name="s")
_CP = pltpu.CompilerParams(needs_layout_passes=False)


def _sortable_u32(x_f32):
    b = lax.bitcast_convert_type(x_f32, jnp.int32)
    m = (b >> 31) | jnp.full((16,), jnp.int32(-(2 ** 31)), jnp.int32)
    return lax.bitcast_convert_type(b ^ m, jnp.uint32)


def _wid():
    return lax.axis_index("c") * 16 + lax.axis_index("s")


_SCHED = ((0, 0, 34560), (0, 1, 34560), (0, 2, 34560), (1, 0, 25920),
          (2, 0, 6480), (3, 0, 1632), (4, 0, 416))


def _hist_body(c0, c1, c2, c3, c4, hist_out, bufa, bufb, hist, sema, semb):
    refs = (c0, c1, c2, c3, c4)
    wid = _wid()
    iota = lax.iota(jnp.int32, 16)
    ones = jnp.ones((16,), jnp.int32)
    zeros = jnp.zeros((16,), jnp.int32)
    bufs = (bufa, bufb)
    sems = (sema, semb)
    u20 = jnp.full((16,), 20, jnp.uint32)

    def _issue(b, i):
        l, k, chunk = _SCHED[i]
        n = _N[l]
        lo = wid * _F[l] + k * chunk
        start = jnp.minimum(lo, n - chunk)
        return pltpu.async_copy(refs[l].at[pl.ds(b * n + start, chunk)],
                                bufs[i % 2].at[pl.ds(0, chunk)], sems[i % 2])

    @pl.loop(0, _B)
    def _batch(b):
        @pl.loop(0, _NBIN // 16)
        def _zero(j):
            hist[pl.ds(j * 16, 16)] = zeros

        copies = [_issue(b, 0)]
        for i, (l, k, chunk) in enumerate(_SCHED):
            if i + 1 < len(_SCHED):
                copies.append(_issue(b, i + 1))
            copies[i].wait()
            buf = bufs[i % 2]
            n = _N[l]
            lo = wid * _F[l] + k * chunk
            hi = jnp.minimum(lo + chunk, jnp.minimum((wid + 1) * _F[l], n))
            start = jnp.minimum(lo, n - chunk)
            if l < 3:
                @pl.loop(0, chunk // 16, unroll=8)
                def _vec(v, buf=buf):
                    x = buf[pl.ds(v * 16, 16)]
                    s = _sortable_u32(x)
                    bin_ = lax.bitcast_convert_type(s >> u20, jnp.int32)
                    plsc.addupdate_scatter(hist, [bin_], ones)
            else:
                @pl.loop(0, chunk // 16, unroll=2)
                def _vec(v, buf=buf, lo=lo, hi=hi, start=start):
                    x = buf[pl.ds(v * 16, 16)]
                    gpos = start + v * 16 + iota
                    mask = (gpos >= lo) & (gpos < hi)
                    s = _sortable_u32(x)
                    bin_ = lax.bitcast_convert_type(s >> u20, jnp.int32)
                    plsc.addupdate_scatter(hist, [bin_], ones, mask=mask)

        pltpu.sync_copy(hist, hist_out.at[b, wid])


def _compact_body(c0, c1, c2, c3, c4, t32, vals_out, idxs_out,
                  bufa, bufb, vbuf, ibuf, tvm, sema, semb):
    refs = (c0, c1, c2, c3, c4)
    wid = _wid()
    iota = lax.iota(jnp.int32, 16)
    ones = jnp.ones((16,), jnp.int32)
    zeros = jnp.zeros((16,), jnp.int32)
    ninf = jnp.full((16,), -jnp.inf, jnp.float32)
    imax = jnp.full((16,), jnp.int32(2 ** 31 - 1), jnp.int32)
    bufs = (bufa, bufb)
    sems = (sema, semb)

    pltpu.sync_copy(t32, tvm)

    def _issue(b, i):
        l, k, chunk = _SCHED[i]
        n = _N[l]
        lo = wid * _F[l] + k * chunk
        start = jnp.minimum(lo, n - chunk)
        return pltpu.async_copy(refs[l].at[pl.ds(b * n + start, chunk)],
                                bufs[i % 2].at[pl.ds(0, chunk)], sems[i % 2])

    @pl.loop(0, _B)
    def _batch(b):
        thr = lax.bitcast_convert_type(
            plsc.load_gather(tvm, [jnp.full((16,), 0, jnp.int32) + b]),
            jnp.uint32)

        copies = [_issue(b, 0)]

        @pl.loop(0, _SLOT // 16)
        def _fill(j):
            vbuf[pl.ds(j * 16, 16)] = ninf
            ibuf[pl.ds(j * 16, 16)] = imax

        cnt = jnp.zeros((16,), jnp.int32)
        for i, (l, k, chunk) in enumerate(_SCHED):
            if i + 1 < len(_SCHED):
                copies.append(_issue(b, i + 1))
            copies[i].wait()
            buf = bufs[i % 2]
            n = _N[l]
            lg = _LG[l]
            lvl90 = _A_OFF[l] * 90
            s2m1 = (1 << lg) - 1
            lo = wid * _F[l] + k * chunk
            hi = jnp.minimum(lo + chunk, jnp.minimum((wid + 1) * _F[l], n))
            start = jnp.minimum(lo, n - chunk)
            masked = l >= 3

            @pl.loop(0, chunk // 16, init_carry=cnt, unroll=8)
            def _vec(v, cnt, buf=buf, lo=lo, hi=hi, start=start, lg=lg,
                     lvl90=lvl90, s2m1=s2m1, masked=masked):
                x = buf[pl.ds(v * 16, 16)]
                gpos = start + v * 16 + iota
                s = _sortable_u32(x)
                keep = s >= thr
                if masked:
                    keep = keep & (gpos >= lo) & (gpos < hi)
                mi = jnp.where(keep, ones, zeros)
                pos = cnt + plsc.cumsum(mi) - 1
                keep = keep & (pos < _SLOT)
                ch = gpos >> lg
                hw = gpos & s2m1
                fidx = hw * 810 + ch + lvl90
                plsc.store_scatter(vbuf, [pos], x, mask=keep)
                plsc.store_scatter(ibuf, [pos], fidx, mask=keep)
                return cnt + plsc.all_reduce_population_count(keep)

            cnt = _vec

        pltpu.sync_copy(vbuf, vals_out.at[b, wid])
        pltpu.sync_copy(ibuf, idxs_out.at[b, wid])


def _gather_body(table, rowids, out, ridx, gidx, rows, obuf, sem):
    base = _wid() * _GCHUNK
    iota = lax.iota(jnp.int32, 16)

    @pl.loop(0, _GCHUNK // 128)
    def _chunk(g):
        pltpu.sync_copy(rowids.at[pl.ds(base + g * 128, 128)], ridx)

        @pl.loop(0, 8)
        def _mkidx(v):
            a = ridx[pl.ds(v * 16, 16)]
            gidx[pl.ds(v * 16, 16)] = a >> 5

        pltpu.async_copy(table.at[gidx], rows, sem).wait()

        @pl.loop(0, 8)
        def _extract(v):
            a = ridx[pl.ds(v * 16, 16)]
            r_local = v * 16 + iota
            col = (a & 31) * 4
            for j in range(4):
                val = plsc.load_gather(rows, [r_local, col + j])
                plsc.store_scatter(obuf, [r_local, iota * 0 + j], val)

        pltpu.sync_copy(obuf, out.at[pl.ds(base + g * 128, 128)])


_pass1 = pl.kernel(
    _hist_body,
    out_type=jax.ShapeDtypeStruct((_B, _NW, _NBIN), jnp.int32),
    mesh=_MESH,
    compiler_params=_CP,
    scratch_types=[pltpu.VMEM((_BUF,), jnp.float32),
                   pltpu.VMEM((_BUF,), jnp.float32),
                   pltpu.VMEM((_NBIN,), jnp.int32),
                   pltpu.SemaphoreType.DMA,
                   pltpu.SemaphoreType.DMA],
)

_pass2 = pl.kernel(
    _compact_body,
    out_type=(jax.ShapeDtypeStruct((_B, _NW, _SLOT), jnp.float32),
              jax.ShapeDtypeStruct((_B, _NW, _SLOT), jnp.int32)),
    mesh=_MESH,
    compiler_params=_CP,
    scratch_types=[pltpu.VMEM((_BUF,), jnp.float32),
                   pltpu.VMEM((_BUF,), jnp.float32),
                   pltpu.VMEM((_SLOT,), jnp.float32),
                   pltpu.VMEM((_SLOT,), jnp.int32),
                   pltpu.VMEM((_B,), jnp.int32),
                   pltpu.SemaphoreType.DMA,
                   pltpu.SemaphoreType.DMA],
)

_pass3 = pl.kernel(
    _gather_body,
    out_type=jax.ShapeDtypeStruct((_B * _KPAD, 4), jnp.float32),
    mesh=_MESH,
    compiler_params=_CP,
    scratch_types=[pltpu.VMEM((128,), jnp.int32),
                   pltpu.VMEM((128,), jnp.int32),
                   pltpu.VMEM((128, 128), jnp.float32),
                   pltpu.VMEM((128, 4), jnp.float32),
                   pltpu.SemaphoreType.DMA],
)


def kernel(cls_out_0, cls_out_1, cls_out_2, cls_out_3, cls_out_4,
           box_out_0, box_out_1, box_out_2, box_out_3, box_out_4):
    cls_in = (cls_out_0, cls_out_1, cls_out_2, cls_out_3, cls_out_4)
    box_in = (box_out_0, box_out_1, box_out_2, box_out_3, box_out_4)
    flats = tuple(c.reshape(-1) for c in cls_in)

    hist = _pass1(*flats)
    counts = hist.sum(axis=1, dtype=jnp.int32)
    csum_ge = jnp.cumsum(counts[:, ::-1], axis=1)[:, ::-1]
    binids = lax.broadcasted_iota(jnp.int32, (_B, _NBIN), 1)
    bstar = jnp.max(jnp.where(csum_ge >= _K, binids, 0), axis=1)
    t32 = lax.bitcast_convert_type(bstar.astype(jnp.uint32) << 20, jnp.int32)

    vals, idxs = _pass2(*flats, t32)
    vflat = vals.reshape(_B, _NW * _SLOT)
    iflat = idxs.reshape(_B, _NW * _SLOT)
    bi = lax.bitcast_convert_type(vflat, jnp.int32)
    kasc = bi ^ ((bi >> 31) & jnp.int32(0x7FFFFFFF))
    _, topidx, topval = lax.sort((-kasc, iflat, vflat), dimension=1, num_keys=2)
    topidx = topidx[:, :_K]
    topval = topval[:, :_K]
    anchors = topidx // _C
    classes = topidx % _C

    table = jnp.concatenate(
        [jnp.transpose(bx, (0, 2, 3, 1)).reshape(_B, -1, 4) for bx in box_in],
        axis=1).reshape(-1, 128)
    apad = jnp.concatenate(
        [anchors, jnp.zeros((_B, _KPAD - _K), jnp.int32)], axis=1)
    rowids = (apad + jnp.arange(_B, dtype=jnp.int32)[:, None] * _TOT_ANCH
              ).reshape(-1)
    boxes = _pass3(table, rowids).reshape(_B, _KPAD, 4)[:, :_K, :]

    return (topval[..., None], boxes, anchors, classes)
